# baseline (speedup 1.0000x reference)
import functools

import jax
import jax.numpy as jnp
from jax import lax
from jax.experimental import pallas as pl
from jax.experimental.pallas import tpu as pltpu
from jax.experimental.pallas import tpu_sc as plsc

N = 10000
E = 320000
NIN = 128
F = 16
NITER = 3

NS = 16
NP = 10240
ROWS_PER_TILE = NP // NS
C = 768
NCHUNK = 27
EDGES_PER_TILE = C * NCHUNK
EP = EDGES_PER_TILE * NS
SUB = C // 128
GROUPS = C // 16
IDX_ROWS_PER_TILE = EDGES_PER_TILE // 128


def _mlp_body(x_ref, w1, b1, w2, b2, w3, b3, w4, b4, w5, b5, w6, b6, o_ref):
    h = x_ref[...]
    for w, b in ((w1, b1), (w2, b2), (w3, b3), (w4, b4), (w5, b5)):
        h = jnp.maximum(
            jnp.dot(h, w[...], preferred_element_type=jnp.float32) + b[...], 0.0)
    o_ref[...] = jnp.dot(h, w6[...], preferred_element_type=jnp.float32) + b6[...]


def _mlp(x, W1, b1, W2, b2, W3, b3, W4, b4, W5, b5, W6, b6):
    R = 2000
    grid = (N // R,)
    full = lambda shp: pl.BlockSpec(shp, lambda i: (0, 0))
    in_specs = [pl.BlockSpec((R, NIN), lambda i: (i, 0))]
    for w, b in ((W1, b1), (W2, b2), (W3, b3), (W4, b4), (W5, b5), (W6, b6)):
        in_specs.append(full(w.shape))
        in_specs.append(full((1, F)))
    return pl.pallas_call(
        _mlp_body,
        grid=grid,
        in_specs=in_specs,
        out_specs=pl.BlockSpec((R, F), lambda i: (i, 0)),
        out_shape=jax.ShapeDtypeStruct((N, F), jnp.float32),
    )(x, W1, b1.reshape(1, F), W2, b2.reshape(1, F), W3, b3.reshape(1, F),
      W4, b4.reshape(1, F), W5, b5.reshape(1, F), W6, b6.reshape(1, F))


def _crf_body(b0_hbm, src_hbm, dst_hbm, ab_hbm, out_hbm,
              h_sp, num_sp, den_sp,
              srcb0, srcb1, srcb2, dstb0, dstb1, dstb2,
              xs0, xs1, xs2, xd0, xd1, xd2, g0, g1, g2,
              nbuf, dbuf, b0_v, ab_v,
              gsem0, gsem1, gsem2, ssem0, ssem1, ssem2):
    srcb = (srcb0, srcb1, srcb2)
    dstb = (dstb0, dstb1, dstb2)
    xs = (xs0, xs1, xs2)
    xd = (xd0, xd1, xd2)
    gg = (g0, g1, g2)
    gsem = (gsem0, gsem1, gsem2)
    ssem = (ssem0, ssem1, ssem2)

    tile = lax.axis_index("s")
    row0 = tile * ROWS_PER_TILE
    iota16 = lax.iota(jnp.int32, 16)
    zrow = jnp.zeros((F,), jnp.float32)

    def _zero_num_den():
        def _zr(i, _):
            xs0[i, :] = zrow
            return _
        lax.fori_loop(0, ROWS_PER_TILE, _zr, None)

        def _zd(i, _):
            g0[pl.ds(i * 16, 16)] = zrow
            return _
        lax.fori_loop(0, ROWS_PER_TILE // 16, _zd, None)
        pltpu.sync_copy(xs0.at[pl.ds(0, ROWS_PER_TILE), :],
                        num_sp.at[pl.ds(row0, ROWS_PER_TILE), :])
        pltpu.sync_copy(g0.at[pl.ds(0, ROWS_PER_TILE)],
                        den_sp.at[pl.ds(row0, ROWS_PER_TILE)])

    pltpu.sync_copy(ab_hbm, ab_v)
    av = ab_v[pl.ds(0, 16)]
    bv = ab_v[pl.ds(16, 16)]

    pltpu.sync_copy(b0_hbm.at[pl.ds(row0, ROWS_PER_TILE), :], b0_v)
    pltpu.sync_copy(b0_v, h_sp.at[pl.ds(row0, ROWS_PER_TILE), :])
    _zero_num_den()
    plsc.subcore_barrier()

    idx_row0 = tile * IDX_ROWS_PER_TILE

    def _idx_copy(c, k):
        r0 = idx_row0 + c * SUB
        pltpu.sync_copy(src_hbm.at[pl.ds(r0, SUB), :], srcb[k])
        pltpu.sync_copy(dst_hbm.at[pl.ds(r0, SUB), :], dstb[k])

    def _gathers(k):
        ds_ = []
        for j in range(SUB):
            ds_.append(pltpu.make_async_copy(
                h_sp.at[srcb[k].at[j]], xs[k].at[pl.ds(j * 128, 128), :],
                gsem[k]))
            ds_.append(pltpu.make_async_copy(
                h_sp.at[dstb[k].at[j]], xd[k].at[pl.ds(j * 128, 128), :],
                gsem[k]))
        return ds_

    def _scatters(k):
        ds_ = []
        for j in range(SUB):
            ds_.append(pltpu.make_async_copy(
                xs[k].at[pl.ds(j * 128, 128), :], num_sp.at[dstb[k].at[j]],
                ssem[k]))
            ds_.append(pltpu.make_async_copy(
                gg[k].at[pl.ds(j * 128, 128)], den_sp.at[dstb[k].at[j]],
                ssem[k]))
        return ds_

    def _compute(k):
        xs_k, xd_k, g_k = xs[k], xd[k], gg[k]

        def _group(j2, _g):
            eidx = j2 * 16 + iota16
            acc = jnp.zeros((16,), jnp.float32)
            vs_list = []
            for f in range(F):
                fv = (iota16 + f) & (F - 1)
                vs = plsc.load_gather(xs_k, [eidx, fv])
                vd = plsc.load_gather(xd_k, [eidx, fv])
                d = vs - vd
                acc = acc + d * d
                vs_list.append(vs)
            g = jnp.exp(acc * (-1.0 / F))
            g_k[pl.ds(j2 * 16, 16)] = g
            for f in range(F):
                fv = (iota16 + f) & (F - 1)
                plsc.store_scatter(xs_k, [eidx, fv], vs_list[f] * g)
            return _g
        lax.fori_loop(0, GROUPS, _group, None)

    for t in range(NITER):
        _idx_copy(0, 0)
        for d in _gathers(0):
            d.start(add=False)

        def _triple(p, _):
            for u in range(3):
                c = p * 3 + u
                nxt = (u + 1) % 3

                @pl.when(c >= 2)
                def _():
                    for d in _scatters(nxt):
                        d.wait()

                @pl.when(c + 1 < NCHUNK)
                def _():
                    _idx_copy(c + 1, nxt)
                    for d in _gathers(nxt):
                        d.start(add=False)

                for d in _gathers(u):
                    d.wait()
                _compute(u)
                for d in _scatters(u):
                    d.start(add=True)
            return _
        lax.fori_loop(0, NCHUNK // 3, _triple, None)

        for d in _scatters((NCHUNK - 2) % 3):
            d.wait()
        for d in _scatters((NCHUNK - 1) % 3):
            d.wait()
        plsc.subcore_barrier()

        pltpu.sync_copy(num_sp.at[pl.ds(row0, ROWS_PER_TILE), :], nbuf)
        pltpu.sync_copy(den_sp.at[pl.ds(row0, ROWS_PER_TILE)], dbuf)

        def _upd(kk, _):
            dvec = dbuf[pl.ds(kk * 16, 16)]
            for r in range(16):
                i = kk * 16 + r
                numr = nbuf[i, :]
                b0r = b0_v[i, :]
                denb = jnp.full((16,), dvec[r], jnp.float32)
                nbuf[i, :] = (av * b0r + bv * numr) / (av + bv * denb)
            return _
        lax.fori_loop(0, ROWS_PER_TILE // 16, _upd, None)

        pltpu.sync_copy(nbuf, h_sp.at[pl.ds(row0, ROWS_PER_TILE), :])
        if t == NITER - 1:
            pltpu.sync_copy(nbuf, out_hbm.at[pl.ds(row0, ROWS_PER_TILE), :])
        else:
            _zero_num_den()
        plsc.subcore_barrier()


_crf = functools.partial(
    pl.kernel,
    _crf_body,
    out_type=jax.ShapeDtypeStruct((NP, F), jnp.float32),
    mesh=plsc.VectorSubcoreMesh(
        core_axis_name="c", subcore_axis_name="s", num_cores=1),
    compiler_params=pltpu.CompilerParams(
        needs_layout_passes=False, use_tc_tiling_on_sc=False),
    scratch_types=(
        [pltpu.VMEM_SHARED((NP, F), jnp.float32),
         pltpu.VMEM_SHARED((NP, F), jnp.float32),
         pltpu.VMEM_SHARED((NP,), jnp.float32)]
        + [pltpu.VMEM((SUB, 128), jnp.int32)] * 6
        + [pltpu.VMEM((C, F), jnp.float32)] * 6
        + [pltpu.VMEM((C,), jnp.float32)] * 3
        + [pltpu.VMEM((ROWS_PER_TILE, F), jnp.float32),
           pltpu.VMEM((ROWS_PER_TILE,), jnp.float32),
           pltpu.VMEM((ROWS_PER_TILE, F), jnp.float32),
           pltpu.VMEM((32,), jnp.float32)]
        + [pltpu.SemaphoreType.DMA] * 6
    ),
)()


def kernel(x, edge_index, W1, b1, W2, b2, W3, b3, W4, b4, W5, b5, W6, b6,
           alpha, beta):
    b0 = _mlp(x, W1, b1, W2, b2, W3, b3, W4, b4, W5, b5, W6, b6)
    b0p = jnp.concatenate([b0, jnp.zeros((NP - N, F), jnp.float32)], axis=0)
    src = edge_index[0]
    dst = edge_index[1]
    pad = EP - E
    pad_src = (jnp.arange(pad, dtype=jnp.int32) * 37) % N
    pad_dst = N + (jnp.arange(pad, dtype=jnp.int32) % (NP - N))
    srcp = jnp.concatenate([src, pad_src]).reshape(EP // 128, 128)
    dstp = jnp.concatenate([dst, pad_dst]).reshape(EP // 128, 128)
    ab = jnp.concatenate([jnp.full((16,), alpha, jnp.float32),
                          jnp.full((16,), beta, jnp.float32)])
    hp = _crf(b0p, srcp, dstp, ab)
    return hp[:N]


# speedup vs baseline: 1.0223x; 1.0223x over previous
import functools

import jax
import jax.numpy as jnp
from jax import lax
from jax.experimental import pallas as pl
from jax.experimental.pallas import tpu as pltpu
from jax.experimental.pallas import tpu_sc as plsc

N = 10000
E = 320000
NIN = 128
F = 16
NITER = 3

NS = 16
NP = 10240
ROWS_PER_TILE = NP // NS
C = 768
NCHUNK = 27
EDGES_PER_TILE = C * NCHUNK
EP = EDGES_PER_TILE * NS
SUB = C // 128
GROUPS = C // 16
IDX_ROWS_PER_TILE = EDGES_PER_TILE // 128


def _mlp_body(x_ref, w1, b1, w2, b2, w3, b3, w4, b4, w5, b5, w6, b6, o_ref):
    h = x_ref[...]
    for w, b in ((w1, b1), (w2, b2), (w3, b3), (w4, b4), (w5, b5)):
        h = jnp.maximum(
            jnp.dot(h, w[...], preferred_element_type=jnp.float32) + b[...], 0.0)
    o_ref[...] = jnp.dot(h, w6[...], preferred_element_type=jnp.float32) + b6[...]


def _mlp(x, W1, b1, W2, b2, W3, b3, W4, b4, W5, b5, W6, b6):
    R = 2000
    grid = (N // R,)
    full = lambda shp: pl.BlockSpec(shp, lambda i: (0, 0))
    in_specs = [pl.BlockSpec((R, NIN), lambda i: (i, 0))]
    for w, b in ((W1, b1), (W2, b2), (W3, b3), (W4, b4), (W5, b5), (W6, b6)):
        in_specs.append(full(w.shape))
        in_specs.append(full((1, F)))
    return pl.pallas_call(
        _mlp_body,
        grid=grid,
        in_specs=in_specs,
        out_specs=pl.BlockSpec((R, F), lambda i: (i, 0)),
        out_shape=jax.ShapeDtypeStruct((N, F), jnp.float32),
    )(x, W1, b1.reshape(1, F), W2, b2.reshape(1, F), W3, b3.reshape(1, F),
      W4, b4.reshape(1, F), W5, b5.reshape(1, F), W6, b6.reshape(1, F))


def _crf_body(b0_hbm, sd_hbm, ab_hbm, out_hbm,
              h_sp, num_sp, den_sp,
              sdb0, sdb1, sdb2,
              xs0, xs1, xs2, xd0, xd1, xd2, g0, g1, g2,
              nbuf, dbuf, b0_v, ab_v,
              gsem0, gsem1, gsem2, ssem0, ssem1, ssem2):
    sdb = (sdb0, sdb1, sdb2)
    xs = (xs0, xs1, xs2)
    xd = (xd0, xd1, xd2)
    gg = (g0, g1, g2)
    gsem = (gsem0, gsem1, gsem2)
    ssem = (ssem0, ssem1, ssem2)

    tile = lax.axis_index("s")
    row0 = tile * ROWS_PER_TILE
    iota16 = lax.iota(jnp.int32, 16)
    zrow = jnp.zeros((F,), jnp.float32)

    def _zero_num_den():
        def _zr(i, _):
            xs0[i, :] = zrow
            return _
        lax.fori_loop(0, ROWS_PER_TILE, _zr, None)

        def _zd(i, _):
            g0[pl.ds(i * 16, 16)] = zrow
            return _
        lax.fori_loop(0, ROWS_PER_TILE // 16, _zd, None)
        pltpu.sync_copy(xs0.at[pl.ds(0, ROWS_PER_TILE), :],
                        num_sp.at[pl.ds(row0, ROWS_PER_TILE), :])
        pltpu.sync_copy(g0.at[pl.ds(0, ROWS_PER_TILE)],
                        den_sp.at[pl.ds(row0, ROWS_PER_TILE)])

    pltpu.sync_copy(ab_hbm, ab_v)
    av = ab_v[pl.ds(0, 16)]
    bv = ab_v[pl.ds(16, 16)]

    pltpu.sync_copy(b0_hbm.at[pl.ds(row0, ROWS_PER_TILE), :], b0_v)
    pltpu.sync_copy(b0_v, h_sp.at[pl.ds(row0, ROWS_PER_TILE), :])
    _zero_num_den()
    plsc.subcore_barrier()

    idx_row0 = tile * IDX_ROWS_PER_TILE

    def _idx_copy(c, k):
        r0 = 2 * (idx_row0 + c * SUB)
        pltpu.sync_copy(sd_hbm.at[pl.ds(r0, 2 * SUB), :], sdb[k])

    def _gathers(k):
        ds_ = []
        for j in range(SUB):
            ds_.append(pltpu.make_async_copy(
                h_sp.at[sdb[k].at[2 * j]], xs[k].at[pl.ds(j * 128, 128), :],
                gsem[k]))
            ds_.append(pltpu.make_async_copy(
                h_sp.at[sdb[k].at[2 * j + 1]], xd[k].at[pl.ds(j * 128, 128), :],
                gsem[k]))
        return ds_

    def _scatters(k):
        ds_ = []
        for j in range(SUB):
            ds_.append(pltpu.make_async_copy(
                xs[k].at[pl.ds(j * 128, 128), :], num_sp.at[sdb[k].at[2 * j + 1]],
                ssem[k]))
            ds_.append(pltpu.make_async_copy(
                gg[k].at[pl.ds(j * 128, 128)], den_sp.at[sdb[k].at[2 * j + 1]],
                ssem[k]))
        return ds_

    def _compute(k):
        xs_k, xd_k, g_k = xs[k], xd[k], gg[k]

        @plsc.parallel_loop(0, GROUPS)
        def _group2(j2):
            eidx = j2 * 16 + iota16
            acc = jnp.zeros((16,), jnp.float32)
            vs_list = []
            for f in range(F):
                fv = (iota16 + f) & (F - 1)
                vs = plsc.load_gather(xs_k, [eidx, fv])
                vd = plsc.load_gather(xd_k, [eidx, fv])
                d = vs - vd
                acc = acc + d * d
                vs_list.append(vs)
            g = jnp.exp(acc * (-1.0 / F))
            g_k[pl.ds(j2 * 16, 16)] = g
            for f in range(F):
                fv = (iota16 + f) & (F - 1)
                plsc.store_scatter(xs_k, [eidx, fv], vs_list[f] * g)

    for t in range(NITER):
        _idx_copy(0, 0)
        for d in _gathers(0):
            d.start(add=False)

        def _triple(p, _):
            for u in range(3):
                c = p * 3 + u
                nxt = (u + 1) % 3

                @pl.when(c >= 2)
                def _():
                    for d in _scatters(nxt):
                        d.wait()

                @pl.when(c + 1 < NCHUNK)
                def _():
                    _idx_copy(c + 1, nxt)
                    for d in _gathers(nxt):
                        d.start(add=False)

                for d in _gathers(u):
                    d.wait()
                _compute(u)
                for d in _scatters(u):
                    d.start(add=True)
            return _
        lax.fori_loop(0, NCHUNK // 3, _triple, None)

        for d in _scatters((NCHUNK - 2) % 3):
            d.wait()
        for d in _scatters((NCHUNK - 1) % 3):
            d.wait()
        plsc.subcore_barrier()

        pltpu.sync_copy(num_sp.at[pl.ds(row0, ROWS_PER_TILE), :], nbuf)
        pltpu.sync_copy(den_sp.at[pl.ds(row0, ROWS_PER_TILE)], dbuf)

        def _upd(kk, _):
            dvec = dbuf[pl.ds(kk * 16, 16)]
            for r in range(16):
                i = kk * 16 + r
                numr = nbuf[i, :]
                b0r = b0_v[i, :]
                denb = jnp.full((16,), dvec[r], jnp.float32)
                nbuf[i, :] = (av * b0r + bv * numr) / (av + bv * denb)
            return _
        lax.fori_loop(0, ROWS_PER_TILE // 16, _upd, None)

        pltpu.sync_copy(nbuf, h_sp.at[pl.ds(row0, ROWS_PER_TILE), :])
        if t == NITER - 1:
            pltpu.sync_copy(nbuf, out_hbm.at[pl.ds(row0, ROWS_PER_TILE), :])
        else:
            _zero_num_den()
        plsc.subcore_barrier()


_crf = functools.partial(
    pl.kernel,
    _crf_body,
    out_type=jax.ShapeDtypeStruct((NP, F), jnp.float32),
    mesh=plsc.VectorSubcoreMesh(
        core_axis_name="c", subcore_axis_name="s", num_cores=1),
    compiler_params=pltpu.CompilerParams(
        needs_layout_passes=False, use_tc_tiling_on_sc=False),
    scratch_types=(
        [pltpu.VMEM_SHARED((NP, F), jnp.float32),
         pltpu.VMEM_SHARED((NP, F), jnp.float32),
         pltpu.VMEM_SHARED((NP,), jnp.float32)]
        + [pltpu.VMEM((2 * SUB, 128), jnp.int32)] * 3
        + [pltpu.VMEM((C, F), jnp.float32)] * 6
        + [pltpu.VMEM((C,), jnp.float32)] * 3
        + [pltpu.VMEM((ROWS_PER_TILE, F), jnp.float32),
           pltpu.VMEM((ROWS_PER_TILE,), jnp.float32),
           pltpu.VMEM((ROWS_PER_TILE, F), jnp.float32),
           pltpu.VMEM((32,), jnp.float32)]
        + [pltpu.SemaphoreType.DMA] * 6
    ),
)()


def kernel(x, edge_index, W1, b1, W2, b2, W3, b3, W4, b4, W5, b5, W6, b6,
           alpha, beta):
    b0 = _mlp(x, W1, b1, W2, b2, W3, b3, W4, b4, W5, b5, W6, b6)
    b0p = jnp.concatenate([b0, jnp.zeros((NP - N, F), jnp.float32)], axis=0)
    src = edge_index[0]
    dst = edge_index[1]
    pad = EP - E
    pad_src = (jnp.arange(pad, dtype=jnp.int32) * 37) % N
    pad_dst = N + (jnp.arange(pad, dtype=jnp.int32) % (NP - N))
    srcp = jnp.concatenate([src, pad_src]).reshape(EP // 128, 128)
    dstp = jnp.concatenate([dst, pad_dst]).reshape(EP // 128, 128)
    sdp = jnp.stack([srcp, dstp], axis=1).reshape(2 * (EP // 128), 128)
    ab = jnp.concatenate([jnp.full((16,), alpha, jnp.float32),
                          jnp.full((16,), beta, jnp.float32)])
    hp = _crf(b0p, sdp, ab)
    return hp[:N]
